# baseline (speedup 1.0000x reference)
import functools

import jax
import jax.numpy as jnp
from jax import lax
from jax.experimental import pallas as pl
from jax.experimental.pallas import tpu as pltpu
from jax.experimental.pallas import tpu_sc as plsc

_VECTOR_DIM = 32
_ROWS_PER_BLOCK = 1024


def _argmin_body(x_ref, c_ref, idx_ref):
    x = x_ref[...]
    c = c_ref[...]
    sim2 = jnp.dot(x * -2.0, c, preferred_element_type=jnp.float32)
    rown = jnp.sum(x * x, axis=1, keepdims=True)
    coln = jnp.sum(c * c, axis=0, keepdims=True)
    dist = (rown + coln) + sim2
    idx_ref[0, 0, :] = jnp.argmin(dist, axis=1).astype(jnp.int32)


def _compute_indices(x, codebook, row_offset, rows):
    v = codebook.shape[1]
    r = _ROWS_PER_BLOCK
    g = rows // r
    off = row_offset // r
    idx3 = pl.pallas_call(
        _argmin_body,
        grid=(g,),
        in_specs=[
            pl.BlockSpec((r, _VECTOR_DIM), lambda i: (i + off, 0)),
            pl.BlockSpec((_VECTOR_DIM, v), lambda i: (0, 0)),
        ],
        out_specs=pl.BlockSpec((1, 1, r), lambda i: (i, 0, 0)),
        out_shape=jax.ShapeDtypeStruct((g, 1, r), jnp.int32),
    )(x, codebook)
    return idx3.reshape(rows)


def _sc_gather_and_loss(table, idx, x, row_offset, rows):
    d = table.shape[1]
    num_cores, num_subcores = 2, 16
    lanes = 16
    nw = num_cores * num_subcores
    b_per_w = rows // nw

    mesh = plsc.VectorSubcoreMesh(core_axis_name="c", subcore_axis_name="s")

    @functools.partial(
        pl.kernel,
        mesh=mesh,
        compiler_params=pltpu.CompilerParams(use_tc_tiling_on_sc=False),
        out_type=[
            jax.ShapeDtypeStruct((rows, d), jnp.float32),
            jax.ShapeDtypeStruct((nw, lanes), jnp.float32),
        ],
        scratch_types=[
            pltpu.VMEM((b_per_w,), jnp.int32),
            pltpu.VMEM((b_per_w, d), jnp.float32),
            pltpu.VMEM((b_per_w, d), jnp.float32),
            pltpu.VMEM((lanes,), jnp.float32),
            pltpu.SemaphoreType.DMA,
        ],
    )
    def gather_kernel(table_hbm, idx_hbm, x_hbm, out_hbm, part_hbm,
                      idx_v, rows_v, x_v, acc_v, sem):
        wid = lax.axis_index("s") * num_cores + lax.axis_index("c")
        base = wid * b_per_w
        cp_x = pltpu.async_copy(
            x_hbm.at[pl.ds(row_offset + base, b_per_w)], x_v, sem)
        pltpu.sync_copy(idx_hbm.at[pl.ds(base, b_per_w)], idx_v)
        pltpu.async_copy(table_hbm.at[idx_v], rows_v, sem).wait()
        cp_x.wait()
        out_cp = pltpu.async_copy(rows_v, out_hbm.at[pl.ds(base, b_per_w)], sem)

        acc_v[...] = jnp.zeros((lanes,), jnp.float32)

        def body(r, _):
            a = acc_v[...]
            for h in range(d // lanes):
                dq = rows_v[r, pl.ds(h * lanes, lanes)] - x_v[r, pl.ds(h * lanes, lanes)]
                a = a + dq * dq
            acc_v[...] = a
            return _

        lax.fori_loop(0, b_per_w, body, 0, unroll=4)
        pltpu.sync_copy(acc_v, part_hbm.at[wid])
        out_cp.wait()

    return gather_kernel(table, idx, x)


def kernel(inputs, quantized_vectors):
    input_shape = inputs.shape
    x = inputs.reshape(-1, _VECTOR_DIM)
    n = x.shape[0]
    ct = quantized_vectors.T
    h = n // 2
    idx_a = _compute_indices(x, quantized_vectors, 0, h)
    qa, pa = _sc_gather_and_loss(ct, idx_a, x, 0, h)
    idx_b = _compute_indices(x, quantized_vectors, h, h)
    qb, pb = _sc_gather_and_loss(ct, idx_b, x, h, h)
    quantized = jnp.concatenate([qa, qb], axis=0)
    vq_loss = (jnp.sum(pa) + jnp.sum(pb)) * (1.25 / (n * _VECTOR_DIM))
    return quantized.reshape(input_shape), vq_loss


# speedup vs baseline: 1.0063x; 1.0063x over previous
import functools

import jax
import jax.numpy as jnp
from jax import lax
from jax.experimental import pallas as pl
from jax.experimental.pallas import tpu as pltpu
from jax.experimental.pallas import tpu_sc as plsc

_VECTOR_DIM = 32
_ROWS_PER_BLOCK = 1024


def _argmin_body(x_ref, c_ref, idx_ref):
    x = x_ref[...]
    c = c_ref[...]
    sim2 = jnp.dot(x * -2.0, c, preferred_element_type=jnp.float32)
    rown = jnp.sum(x * x, axis=1, keepdims=True)
    coln = jnp.sum(c * c, axis=0, keepdims=True)
    dist = (rown + coln) + sim2
    idx_ref[0, 0, :] = jnp.argmin(dist, axis=1).astype(jnp.int32)


def _compute_indices(x, codebook):
    n, _ = x.shape
    v = codebook.shape[1]
    r = _ROWS_PER_BLOCK
    g = n // r
    idx3 = pl.pallas_call(
        _argmin_body,
        grid=(g,),
        in_specs=[
            pl.BlockSpec((r, _VECTOR_DIM), lambda i: (i, 0)),
            pl.BlockSpec((_VECTOR_DIM, v), lambda i: (0, 0)),
        ],
        out_specs=pl.BlockSpec((1, 1, r), lambda i: (i, 0, 0)),
        out_shape=jax.ShapeDtypeStruct((g, 1, r), jnp.int32),
    )(x, codebook)
    return idx3.reshape(n)


def _sc_gather_and_loss(table, idx, x):
    n = idx.shape[0]
    d = table.shape[1]
    num_cores, num_subcores = 2, 16
    lanes = 16
    nw = num_cores * num_subcores
    b_per_w = n // nw

    mesh = plsc.VectorSubcoreMesh(core_axis_name="c", subcore_axis_name="s")

    @functools.partial(
        pl.kernel,
        mesh=mesh,
        compiler_params=pltpu.CompilerParams(use_tc_tiling_on_sc=False),
        out_type=[
            jax.ShapeDtypeStruct((n, d), jnp.float32),
            jax.ShapeDtypeStruct((nw, lanes), jnp.float32),
        ],
        scratch_types=[
            pltpu.VMEM((b_per_w,), jnp.int32),
            pltpu.VMEM((b_per_w, d), jnp.float32),
            pltpu.VMEM((b_per_w, d), jnp.float32),
            pltpu.VMEM((lanes,), jnp.float32),
            pltpu.SemaphoreType.DMA,
        ],
    )
    def gather_kernel(table_hbm, idx_hbm, x_hbm, out_hbm, part_hbm,
                      idx_v, rows_v, x_v, acc_v, sem):
        wid = lax.axis_index("s") * num_cores + lax.axis_index("c")
        base = wid * b_per_w
        pltpu.sync_copy(idx_hbm.at[pl.ds(base, b_per_w)], idx_v)
        cp_x = pltpu.async_copy(x_hbm.at[pl.ds(base, b_per_w)], x_v, sem)
        pltpu.async_copy(table_hbm.at[idx_v], rows_v, sem).wait()
        cp_x.wait()
        out_cp = pltpu.async_copy(rows_v, out_hbm.at[pl.ds(base, b_per_w)], sem)

        acc_v[...] = jnp.zeros((lanes,), jnp.float32)

        def body(r, _):
            a = acc_v[...]
            for h in range(d // lanes):
                dq = rows_v[r, pl.ds(h * lanes, lanes)] - x_v[r, pl.ds(h * lanes, lanes)]
                a = a + dq * dq
            acc_v[...] = a
            return _

        lax.fori_loop(0, b_per_w, body, 0, unroll=4)
        pltpu.sync_copy(acc_v, part_hbm.at[wid])
        out_cp.wait()

    return gather_kernel(table, idx, x)


def kernel(inputs, quantized_vectors):
    input_shape = inputs.shape
    x = inputs.reshape(-1, _VECTOR_DIM)
    n = x.shape[0]
    idx = _compute_indices(x, quantized_vectors)
    quantized, partials = _sc_gather_and_loss(quantized_vectors.T, idx, x)
    vq_loss = jnp.sum(partials) * (1.25 / (n * _VECTOR_DIM))
    return quantized.reshape(input_shape), vq_loss
